# Initial kernel scaffold; baseline (speedup 1.0000x reference)
import functools

import jax
import jax.numpy as jnp
from jax import lax
from jax.experimental import pallas as pl
from jax.experimental.pallas import tpu as pltpu
from jax.experimental.pallas import tpu_sc as plsc

N = 10000
E = 320000
D = 128

NC = 2
NS = 16
TILE_E = E // NS
CHUNK = 80
NCH = TILE_E // CHUNK
ROWS_PT = N // NS
NPAD_CNT = 10240
CNT_PT = NPAD_CNT // NS


def _sc_body(nodes_hbm, gidx_hbm, sidx_hbm, zrow_hbm, zcnt_hbm,
             acc_out, cnt_out,
             acc_s, cnt_s, gl, sl, buf, ones, sem):
    c = lax.axis_index("c")
    s = lax.axis_index("s")

    pltpu.sync_copy(gidx_hbm.at[c, s], gl)
    pltpu.sync_copy(sidx_hbm.at[c, s], sl)

    for i in range(CHUNK // 16):
        ones[pl.ds(i * 16, 16)] = jnp.ones((16,), jnp.float32)

    pltpu.sync_copy(zrow_hbm, acc_s.at[pl.ds(s * ROWS_PT, ROWS_PT)])
    pltpu.sync_copy(zcnt_hbm, cnt_s.at[pl.ds(s * CNT_PT, CNT_PT)])
    plsc.subcore_barrier()

    def body(j, carry):
        pltpu.async_copy(nodes_hbm.at[gl.at[j]], buf, sem).wait()
        pltpu.sync_copy(buf, acc_s.at[sl.at[j]], add=True)
        pltpu.sync_copy(ones, cnt_s.at[sl.at[j]], add=True)
        return carry

    lax.fori_loop(0, NCH, body, 0)
    plsc.subcore_barrier()

    pltpu.sync_copy(acc_s.at[pl.ds(s * ROWS_PT, ROWS_PT)],
                    acc_out.at[c, pl.ds(s * ROWS_PT, ROWS_PT)])
    pltpu.sync_copy(cnt_s.at[pl.ds(s * CNT_PT, CNT_PT)],
                    cnt_out.at[c, pl.ds(s * CNT_PT, CNT_PT)])


_sc_aggregate = pl.kernel(
    _sc_body,
    out_type=(
        jax.ShapeDtypeStruct((NC, N, D), jnp.float32),
        jax.ShapeDtypeStruct((NC, NPAD_CNT), jnp.float32),
    ),
    mesh=plsc.VectorSubcoreMesh(core_axis_name="c", subcore_axis_name="s"),
    scratch_types=[
        pltpu.VMEM_SHARED((N, D), jnp.float32),
        pltpu.VMEM_SHARED((NPAD_CNT,), jnp.float32),
        pltpu.VMEM((NCH, CHUNK), jnp.int32),
        pltpu.VMEM((NCH, CHUNK), jnp.int32),
        pltpu.VMEM((CHUNK, D), jnp.float32),
        pltpu.VMEM((CHUNK,), jnp.float32),
        pltpu.SemaphoreType.DMA,
    ],
)


ROWS_TC = 500


def _tc_body(nodes_ref, a1_ref, a2_ref, cnt_ref, w0_ref, w1_ref, w2_ref,
             out_ref):
    dn = (((1,), (1,)), ((), ()))
    cnt = jnp.sum(cnt_ref[...], axis=1, keepdims=True)
    denom = jnp.maximum(cnt, 1.0)
    msg = (lax.dot_general(a1_ref[...], w1_ref[...], dn,
                           preferred_element_type=jnp.float32)
           + lax.dot_general(a2_ref[...], w2_ref[...], dn,
                             preferred_element_type=jnp.float32)) / denom
    self_t = lax.dot_general(nodes_ref[...], w0_ref[...], dn,
                             preferred_element_type=jnp.float32)
    out_ref[...] = jnp.maximum(self_t + msg, 0.0)


def _tc_combine(nodes, a1, a2, cnt2, w0, w1, w2):
    row_spec = pl.BlockSpec((ROWS_TC, D), lambda i: (i, 0))
    w_spec = pl.BlockSpec((D, D), lambda i: (0, 0))
    return pl.pallas_call(
        _tc_body,
        grid=(N // ROWS_TC,),
        in_specs=[row_spec, row_spec, row_spec,
                  pl.BlockSpec((ROWS_TC, NC), lambda i: (i, 0)),
                  w_spec, w_spec, w_spec],
        out_specs=row_spec,
        out_shape=jax.ShapeDtypeStruct((N, D), jnp.float32),
    )(nodes, a1, a2, cnt2, w0, w1, w2)


def kernel(nodes, edges, W0, W1, W2):
    edges = edges.astype(jnp.int32)
    src = edges[0]
    dst = edges[1]
    gidx = jnp.stack([src, dst]).reshape(NC, NS, NCH, CHUNK)
    sidx = jnp.stack([dst, src]).reshape(NC, NS, NCH, CHUNK)
    zrow = jnp.zeros((ROWS_PT, D), jnp.float32)
    zcnt = jnp.zeros((CNT_PT,), jnp.float32)
    acc, cnt = _sc_aggregate(nodes, gidx, sidx, zrow, zcnt)
    cnt2 = cnt[:, :N].T
    return _tc_combine(nodes, acc[0], acc[1], cnt2, W0, W1, W2)


# speedup vs baseline: 9.3140x; 9.3140x over previous
import functools

import jax
import jax.numpy as jnp
from jax import lax
from jax.experimental import pallas as pl
from jax.experimental.pallas import tpu as pltpu
from jax.experimental.pallas import tpu_sc as plsc

N = 10000
E = 320000
D = 128

NC = 2
NS = 16
TILE_E = E // NS
CHUNK = 80
NCH = TILE_E // CHUNK
BCH = 25
NPAD = 10240
ROWS_PT = NPAD // NS
CNT_PT = NPAD // NS


def _sc_body(nodes_hbm, gidx_hbm, sidx_hbm, zrow_hbm, zcnt_hbm,
             acc_out, cnt_out,
             acc_s, cnt_s, gl, sl, buf, ones, sem):
    c = lax.axis_index("c")
    s = lax.axis_index("s")

    for i in range(CHUNK // 16):
        ones[pl.ds(i * 16, 16)] = jnp.ones((16,), jnp.float32)

    pltpu.sync_copy(zrow_hbm, acc_s.at[pl.ds(s * ROWS_PT, ROWS_PT)])
    pltpu.sync_copy(zcnt_hbm, cnt_s.at[pl.ds(s * CNT_PT, CNT_PT)])
    plsc.subcore_barrier()

    def body(b, carry):
        pltpu.sync_copy(gidx_hbm.at[c, s, pl.ds(b * BCH, BCH)], gl)
        pltpu.sync_copy(sidx_hbm.at[c, s, pl.ds(b * BCH, BCH)], sl)

        def inner(j, carry2):
            pltpu.async_copy(nodes_hbm.at[gl.at[j]], buf, sem).wait()
            pltpu.sync_copy(buf, acc_s.at[sl.at[j]], add=True)
            pltpu.sync_copy(ones, cnt_s.at[sl.at[j]], add=True)
            return carry2

        lax.fori_loop(0, BCH, inner, 0)
        return carry

    lax.fori_loop(0, NCH // BCH, body, 0)
    plsc.subcore_barrier()

    pltpu.sync_copy(acc_s.at[pl.ds(s * ROWS_PT, ROWS_PT)],
                    acc_out.at[c, pl.ds(s * ROWS_PT, ROWS_PT)])
    pltpu.sync_copy(cnt_s.at[pl.ds(s * CNT_PT, CNT_PT)],
                    cnt_out.at[c, pl.ds(s * CNT_PT, CNT_PT)])


_sc_aggregate = pl.kernel(
    _sc_body,
    out_type=(
        jax.ShapeDtypeStruct((NC, NPAD, D), jnp.float32),
        jax.ShapeDtypeStruct((NC, NPAD), jnp.float32),
    ),
    mesh=plsc.VectorSubcoreMesh(core_axis_name="c", subcore_axis_name="s"),
    scratch_types=[
        pltpu.VMEM_SHARED((NPAD, D), jnp.float32),
        pltpu.VMEM_SHARED((NPAD,), jnp.float32),
        pltpu.VMEM((BCH, CHUNK), jnp.int32),
        pltpu.VMEM((BCH, CHUNK), jnp.int32),
        pltpu.VMEM((CHUNK, D), jnp.float32),
        pltpu.VMEM((CHUNK,), jnp.float32),
        pltpu.SemaphoreType.DMA,
    ],
    compiler_params=pltpu.CompilerParams(use_tc_tiling_on_sc=False),
)


ROWS_TC = 400


def _tc_body(nodes_ref, a1_ref, a2_ref, cnt_ref, w0_ref, w1_ref, w2_ref,
             out_ref):
    dn = (((1,), (1,)), ((), ()))
    cnt = jnp.sum(cnt_ref[...], axis=1, keepdims=True)
    denom = jnp.maximum(cnt, 1.0)
    msg = (lax.dot_general(a1_ref[...], w1_ref[...], dn,
                           preferred_element_type=jnp.float32)
           + lax.dot_general(a2_ref[...], w2_ref[...], dn,
                             preferred_element_type=jnp.float32)) / denom
    self_t = lax.dot_general(nodes_ref[...], w0_ref[...], dn,
                             preferred_element_type=jnp.float32)
    out_ref[...] = jnp.maximum(self_t + msg, 0.0)


def _tc_combine(nodes, a1, a2, cnt2, w0, w1, w2):
    row_spec = pl.BlockSpec((ROWS_TC, D), lambda i: (i, 0))
    w_spec = pl.BlockSpec((D, D), lambda i: (0, 0))
    return pl.pallas_call(
        _tc_body,
        grid=(N // ROWS_TC,),
        in_specs=[row_spec, row_spec, row_spec,
                  pl.BlockSpec((ROWS_TC, NC), lambda i: (i, 0)),
                  w_spec, w_spec, w_spec],
        out_specs=row_spec,
        out_shape=jax.ShapeDtypeStruct((N, D), jnp.float32),
    )(nodes, a1, a2, cnt2, w0, w1, w2)


def kernel(nodes, edges, W0, W1, W2):
    edges = edges.astype(jnp.int32)
    src = edges[0]
    dst = edges[1]
    gidx = jnp.stack([src, dst]).reshape(NC, NS, NCH, CHUNK)
    sidx = jnp.stack([dst, src]).reshape(NC, NS, NCH, CHUNK)
    zrow = jnp.zeros((ROWS_PT, D), jnp.float32)
    zcnt = jnp.zeros((CNT_PT,), jnp.float32)
    acc, cnt = _sc_aggregate(nodes, gidx, sidx, zrow, zcnt)
    acc = acc[:, :N]
    cnt2 = cnt[:, :N].T
    return _tc_combine(nodes, acc[0], acc[1], cnt2, W0, W1, W2)


# speedup vs baseline: 15.3812x; 1.6514x over previous
import jax
import jax.numpy as jnp
from jax import lax
from jax.experimental import pallas as pl
from jax.experimental.pallas import tpu as pltpu
from jax.experimental.pallas import tpu_sc as plsc

N = 10000
E = 320000
D = 128

NC = 2
NS = 16
TILE_E = E // NS
CHUNK = 80
NCH = TILE_E // CHUNK
BCH = 25
NBLK = NCH // BCH
NPAD = 10240
ROWS_PT = NPAD // NS


def _sc_body(nodes_hbm, eidx_hbm, zrow_hbm, zcnt_hbm,
             acc1_out, acc2_out, cnt_out,
             acc_s, cnt_s, gl, sl, buf0, buf1, ones, sem0, sem1):
    c = lax.axis_index("c")
    s = lax.axis_index("s")

    for i in range(CHUNK // 16):
        ones[pl.ds(i * 16, 16)] = jnp.ones((16,), jnp.float32)

    pltpu.sync_copy(zrow_hbm, acc_s.at[pl.ds(s * ROWS_PT, ROWS_PT)])
    pltpu.sync_copy(zcnt_hbm, cnt_s.at[pl.ds(s * ROWS_PT, ROWS_PT)])
    plsc.subcore_barrier()

    def wait0():
        pltpu.make_async_copy(nodes_hbm.at[gl.at[0]], buf0, sem0).wait()

    def wait1():
        pltpu.make_async_copy(nodes_hbm.at[gl.at[0]], buf1, sem1).wait()

    def scat0(j):
        pltpu.sync_copy(buf0, acc_s.at[sl.at[j]], add=True)
        pltpu.sync_copy(ones, cnt_s.at[sl.at[j]], add=True)

    def scat1(j):
        pltpu.sync_copy(buf1, acc_s.at[sl.at[j]], add=True)
        pltpu.sync_copy(ones, cnt_s.at[sl.at[j]], add=True)

    def block(b, carry):
        pltpu.sync_copy(eidx_hbm.at[c, s, pl.ds(b * BCH, BCH)], gl)
        pltpu.sync_copy(eidx_hbm.at[1 - c, s, pl.ds(b * BCH, BCH)], sl)
        pltpu.async_copy(nodes_hbm.at[gl.at[0]], buf0, sem0)

        def inner(j2, carry2):
            j = 2 * j2
            pltpu.async_copy(nodes_hbm.at[gl.at[j + 1]], buf1, sem1)
            wait0()
            scat0(j)
            pltpu.async_copy(nodes_hbm.at[gl.at[j + 2]], buf0, sem0)
            wait1()
            scat1(j + 1)
            return carry2

        lax.fori_loop(0, (BCH - 1) // 2, inner, 0)
        wait0()
        scat0(BCH - 1)
        return carry

    lax.fori_loop(0, NBLK, block, 0)
    plsc.subcore_barrier()

    sl_pt = pl.ds(s * ROWS_PT, ROWS_PT)

    @pl.when(c == 0)
    def _():
        pltpu.sync_copy(acc_s.at[sl_pt], acc1_out.at[sl_pt])

    @pl.when(c == 1)
    def _():
        pltpu.sync_copy(acc_s.at[sl_pt], acc2_out.at[sl_pt])

    pltpu.sync_copy(cnt_s.at[sl_pt], cnt_out.at[c, sl_pt])


_sc_aggregate = pl.kernel(
    _sc_body,
    out_type=(
        jax.ShapeDtypeStruct((NPAD, D), jnp.float32),
        jax.ShapeDtypeStruct((NPAD, D), jnp.float32),
        jax.ShapeDtypeStruct((NC, NPAD), jnp.float32),
    ),
    mesh=plsc.VectorSubcoreMesh(core_axis_name="c", subcore_axis_name="s"),
    scratch_types=[
        pltpu.VMEM_SHARED((NPAD, D), jnp.float32),
        pltpu.VMEM_SHARED((NPAD,), jnp.float32),
        pltpu.VMEM((BCH, CHUNK), jnp.int32),
        pltpu.VMEM((BCH, CHUNK), jnp.int32),
        pltpu.VMEM((CHUNK, D), jnp.float32),
        pltpu.VMEM((CHUNK, D), jnp.float32),
        pltpu.VMEM((CHUNK,), jnp.float32),
        pltpu.SemaphoreType.DMA,
        pltpu.SemaphoreType.DMA,
    ],
    compiler_params=pltpu.CompilerParams(use_tc_tiling_on_sc=False),
)


ROWS_TC = 400


def _tc_body(nodes_ref, a1_ref, a2_ref, cnt_ref, w0_ref, w1_ref, w2_ref,
             out_ref):
    dn = (((1,), (1,)), ((), ()))
    cnt = jnp.sum(cnt_ref[...], axis=1, keepdims=True)
    denom = jnp.maximum(cnt, 1.0)
    msg = (lax.dot_general(a1_ref[...], w1_ref[...], dn,
                           preferred_element_type=jnp.float32)
           + lax.dot_general(a2_ref[...], w2_ref[...], dn,
                             preferred_element_type=jnp.float32)) / denom
    self_t = lax.dot_general(nodes_ref[...], w0_ref[...], dn,
                             preferred_element_type=jnp.float32)
    out_ref[...] = jnp.maximum(self_t + msg, 0.0)


def _tc_combine(nodes, a1, a2, cnt2, w0, w1, w2):
    row_spec = pl.BlockSpec((ROWS_TC, D), lambda i: (i, 0))
    w_spec = pl.BlockSpec((D, D), lambda i: (0, 0))
    return pl.pallas_call(
        _tc_body,
        grid=(N // ROWS_TC,),
        in_specs=[row_spec, row_spec, row_spec,
                  pl.BlockSpec((ROWS_TC, NC), lambda i: (i, 0)),
                  w_spec, w_spec, w_spec],
        out_specs=row_spec,
        out_shape=jax.ShapeDtypeStruct((N, D), jnp.float32),
    )(nodes, a1, a2, cnt2, w0, w1, w2)


def kernel(nodes, edges, W0, W1, W2):
    eidx = edges.astype(jnp.int32).reshape(NC, NS, NBLK * BCH, CHUNK)
    zrow = jnp.zeros((ROWS_PT, D), jnp.float32)
    zcnt = jnp.zeros((ROWS_PT,), jnp.float32)
    a1, a2, cnt = _sc_aggregate(nodes, eidx, zrow, zcnt)
    return _tc_combine(nodes, a1, a2, cnt.T, W0, W1, W2)
